# baseline (speedup 1.0000x reference)
import functools

import jax
import jax.numpy as jnp
from jax import lax
from jax.experimental import pallas as pl
from jax.experimental.pallas import tpu as pltpu
from jax.experimental.pallas import tpu_sc as plsc

B_, S_, D_ = 1024, 200, 4
EMBED = 64
VOCAB = 1000000
N = B_ * S_
NW = 32
ROWS_W = N // NW
C = 128
G = C * D_
SUB = 128
NSUB = G // SUB
SUPER = 2
NSUPER = ROWS_W // (C * SUPER)
LANES = 16
NGRP = C // LANES

_mesh = plsc.VectorSubcoreMesh(core_axis_name="c", subcore_axis_name="s")


@functools.partial(
    pl.kernel,
    out_type=jax.ShapeDtypeStruct((N // 2, 2 * EMBED), jnp.float32),
    mesh=_mesh,
    compiler_params=pltpu.CompilerParams(needs_layout_passes=False),
    scratch_types=[
        pltpu.VMEM((SUPER * NSUB, SUB), jnp.int32),
        pltpu.VMEM((SUPER * NSUB, SUB), jnp.int32),
        pltpu.VMEM((G, 2 * EMBED), jnp.float32),
        pltpu.VMEM((C // 2, 2 * EMBED), jnp.float32),
        pltpu.SemaphoreType.DMA,
    ],
)
def _emb_pool(pidx_hbm, poff_hbm, table2_hbm, out_hbm, pidx_v, poff_v, pbuf,
              obuf, sem):
    wid = lax.axis_index("s") * 2 + lax.axis_index("c")
    base = wid * ROWS_W

    iota = lax.iota(jnp.int32, LANES)
    alt64 = (iota & 1) * EMBED
    half = iota >> 1

    def super_chunk(go, carry):
        rbase = base + go * (C * SUPER)
        irow = pl.multiple_of(rbase * D_ // SUB, SUPER * NSUB)
        pltpu.sync_copy(pidx_hbm.at[pl.ds(irow, SUPER * NSUB)], pidx_v)
        pltpu.sync_copy(poff_hbm.at[pl.ds(irow, SUPER * NSUB)], poff_v)

        for c in range(SUPER):
            descs = [
                pltpu.async_copy(
                    table2_hbm.at[pidx_v.at[c * NSUB + j]],
                    pbuf.at[pl.ds(j * SUB, SUB)],
                    sem,
                )
                for j in range(NSUB)
            ]
            for d in descs:
                d.wait()

            def group(g, carry2):
                m = g * LANES
                rowv = [4 * (m + iota) + d for d in range(D_)]
                offv = [poff_v[c * NSUB + d, pl.ds(m, LANES)]
                        for d in range(D_)]
                orow = (m + iota) >> 1

                def elem(e, carry3):
                    acc = plsc.load_gather(pbuf, [rowv[0], offv[0] + e])
                    for d in range(1, D_):
                        acc = acc + plsc.load_gather(pbuf,
                                                     [rowv[d], offv[d] + e])
                    plsc.store_scatter(obuf, [orow, alt64 + e], acc)
                    return carry3

                lax.fori_loop(0, EMBED, elem, 0, unroll=4)
                return carry2

            lax.fori_loop(0, NGRP, group, 0)
            obase = pl.multiple_of((rbase + c * C) // 2, EMBED)
            pltpu.sync_copy(obuf, out_hbm.at[pl.ds(obase, C // 2)])
        return carry

    lax.fori_loop(0, NSUPER, super_chunk, 0)


def kernel(input, table):
    b, s, d = input.shape
    flat = input.reshape(-1)
    pidx = (flat >> 1).reshape(-1, SUB)
    poff = ((input.reshape(-1, C, D_) & 1) * EMBED).transpose(0, 2, 1)
    poff = poff.reshape(-1, SUB)
    table2 = table.reshape(VOCAB // 2, 2 * EMBED)
    out = _emb_pool(pidx, poff, table2)
    return out.reshape(b, s, EMBED)


# speedup vs baseline: 2.0061x; 2.0061x over previous
import functools

import jax
import jax.numpy as jnp
from jax import lax
from jax.experimental import pallas as pl
from jax.experimental.pallas import tpu as pltpu
from jax.experimental.pallas import tpu_sc as plsc

B_, S_, D_ = 1024, 200, 4
EMBED = 64
N = B_ * S_
NW = 32
ROWS_W = N // NW
C = 256
G = C * D_
NCHUNK = ROWS_W // C
SUB = 128
NSUB = G // SUB
LANES = 16
QE = EMBED // LANES

_mesh = plsc.VectorSubcoreMesh(core_axis_name="c", subcore_axis_name="s")


@functools.partial(
    pl.kernel,
    out_type=jax.ShapeDtypeStruct((N // 2, 2 * EMBED), jnp.float32),
    mesh=_mesh,
    compiler_params=pltpu.CompilerParams(use_tc_tiling_on_sc=False),
    scratch_types=[
        pltpu.VMEM((NSUB, SUB), jnp.int32),
        pltpu.VMEM((G, EMBED), jnp.float32),
        pltpu.VMEM((C // 2, 2 * EMBED), jnp.float32),
        pltpu.SemaphoreType.DMA,
    ],
)
def _emb_pool(idx_hbm, table_hbm, out_hbm, idx_v, gbuf, obuf, sem):
    wid = lax.axis_index("s") * 2 + lax.axis_index("c")
    base = wid * ROWS_W

    def chunk(g, carry):
        rbase = base + g * C
        irow = pl.multiple_of(rbase * D_ // SUB, NSUB)
        pltpu.sync_copy(idx_hbm.at[pl.ds(irow, NSUB)], idx_v)
        descs = [
            pltpu.async_copy(
                table_hbm.at[idx_v.at[j]],
                gbuf.at[pl.ds(j * SUB, SUB)],
                sem,
            )
            for j in range(NSUB)
        ]
        for d in descs:
            d.wait()

        def pool(n, carry2):
            r = n * D_
            for q in range(QE):
                sl = pl.ds(q * LANES, LANES)
                acc = gbuf[r, sl]
                for k in range(1, D_):
                    acc = acc + gbuf[r + k, sl]
                obuf[n >> 1, pl.ds((n & 1) * EMBED + q * LANES, LANES)] = acc
            return carry2

        lax.fori_loop(0, C, pool, 0, unroll=2)
        pltpu.sync_copy(obuf, out_hbm.at[pl.ds(rbase // 2, C // 2)])
        return carry

    lax.fori_loop(0, NCHUNK, chunk, 0)


def kernel(input, table):
    b, s, d = input.shape
    flat_idx = input.reshape(b * s * d // SUB, SUB)
    out = _emb_pool(flat_idx, table)
    return out.reshape(b, s, EMBED)
